# baseline (speedup 1.0000x reference)
import functools

import jax
import jax.numpy as jnp
from jax import lax
from jax.experimental import pallas as pl
from jax.experimental.pallas import tpu as pltpu
from jax.experimental.pallas import tpu_sc as plsc

B = 16384
NC = 2
NS = 16
L = 16
NW = NC * NS
N_FLAT = B * 5
L_FLAT = B
N_PER_W = N_FLAT // NW
L_PER_W = L_FLAT // NW
N_STEPS = N_PER_W // L
L_STEPS = L_PER_W // L
N_CHUNKS = 4
C_STEPS = N_STEPS // N_CHUNKS
C_ELEMS = C_STEPS * L

_mesh = plsc.VectorSubcoreMesh(core_axis_name="c", subcore_axis_name="s")


_GDN = lax.GatherDimensionNumbers(
    offset_dims=(), collapsed_slice_dims=(0,), start_index_map=(0,))


def _perm(x, idx):
    return lax.gather(x, idx[:, None], _GDN, slice_sizes=(1,),
                      mode=lax.GatherScatterMode.PROMISE_IN_BOUNDS)


def _lane_sum(v):
    lane = lax.iota(jnp.int32, L)
    for k in (8, 4, 2, 1):
        v = v + _perm(v, jnp.bitwise_xor(lane, k))
    return v


def _softmax_scatter(raw_ref, tab_ref, n_vregs, n_valid):
    lane = lax.iota(jnp.int32, L)
    es = []
    esum = jnp.zeros((L,), jnp.float32)
    for i in range(n_vregs):
        v = raw_ref[pl.ds(i * L, L)]
        valid = lane + (i * L) < n_valid
        e = jnp.where(valid, jnp.exp(v), jnp.zeros((L,), jnp.float32))
        es.append((e, valid))
        esum = esum + e
    r = 1.0 / _lane_sum(esum)
    for i, (e, valid) in enumerate(es):
        plsc.store_scatter(tab_ref, [lane + (i * L + 1)], e * r, mask=valid)


@functools.partial(
    pl.kernel,
    out_type=(
        jax.ShapeDtypeStruct((N_FLAT,), jnp.float32),
        jax.ShapeDtypeStruct((L_FLAT,), jnp.float32),
    ),
    mesh=_mesh,
    scratch_types=[
        pltpu.VMEM((64,), jnp.float32),
        pltpu.VMEM((16,), jnp.float32),
        pltpu.VMEM((64,), jnp.float32),
        pltpu.VMEM((16,), jnp.float32),
        pltpu.VMEM((N_PER_W,), jnp.int32),
        pltpu.VMEM((N_PER_W,), jnp.float32),
        pltpu.VMEM((L_PER_W,), jnp.int32),
        pltpu.VMEM((L_PER_W,), jnp.float32),
        pltpu.SemaphoreType.DMA,
        pltpu.SemaphoreType.DMA,
        pltpu.SemaphoreType.DMA,
    ],
    compiler_params=pltpu.CompilerParams(needs_layout_passes=False),
)
def _gather_probs(gn_hbm, lk_hbm, ln_hbm, ll_hbm, out_n_hbm, out_l_hbm,
                  nraw, lraw, ntab, ltab, nidx, nout, lidx, lout,
                  sem_t, sem_i, sem_o):
    wid = lax.axis_index("s") * NC + lax.axis_index("c")
    nbase = wid * N_PER_W
    lbase = wid * L_PER_W

    ln_c = pltpu.async_copy(ln_hbm, nraw.at[pl.ds(0, 49)], sem_t)
    ll_c = pltpu.async_copy(ll_hbm, lraw.at[pl.ds(0, 10)], sem_t)
    ni_c = pltpu.async_copy(gn_hbm.at[pl.ds(nbase, N_PER_W)], nidx, sem_i)
    li_c = pltpu.async_copy(lk_hbm.at[pl.ds(lbase, L_PER_W)], lidx, sem_i)

    ln_c.wait()
    ll_c.wait()
    _softmax_scatter(nraw, ntab, 4, 49)
    _softmax_scatter(lraw, ltab, 1, 10)

    ni_c.wait()
    li_c.wait()

    def nbody(i, carry):
        idx = nidx[pl.ds(i * L, L)]
        nout[pl.ds(i * L, L)] = plsc.load_gather(ntab, [idx])
        return carry

    out_copies = []
    for c in range(N_CHUNKS):
        lax.fori_loop(c * C_STEPS, (c + 1) * C_STEPS, nbody, 0, unroll=8)
        out_copies.append(pltpu.async_copy(
            nout.at[pl.ds(c * C_ELEMS, C_ELEMS)],
            out_n_hbm.at[pl.ds(nbase + c * C_ELEMS, C_ELEMS)], sem_o))

    def lbody(i, carry):
        idx = lidx[pl.ds(i * L, L)]
        lout[pl.ds(i * L, L)] = plsc.load_gather(ltab, [idx])
        return carry

    lax.fori_loop(0, L_STEPS, lbody, 0, unroll=8)
    out_copies.append(
        pltpu.async_copy(lout, out_l_hbm.at[pl.ds(lbase, L_PER_W)], sem_o))

    for c in out_copies:
        c.wait()


def kernel(good_normal, lucky, log_normal_probs, log_lucky_probs):
    out_n, out_l = _gather_probs(
        good_normal.reshape(-1), lucky.reshape(-1),
        log_normal_probs, log_lucky_probs)
    return out_n.reshape(B, 5), out_l.reshape(B, 1)


# speedup vs baseline: 1.0015x; 1.0015x over previous
import functools

import jax
import jax.numpy as jnp
from jax import lax
from jax.experimental import pallas as pl
from jax.experimental.pallas import tpu as pltpu
from jax.experimental.pallas import tpu_sc as plsc

B = 16384
NC = 2
NS = 16
L = 16
NW = NC * NS
N_FLAT = B * 5
L_FLAT = B
N_PER_W = N_FLAT // NW
L_PER_W = L_FLAT // NW
N_STEPS = N_PER_W // L
L_STEPS = L_PER_W // L
N_CHUNKS = 4
C_STEPS = N_STEPS // N_CHUNKS
C_ELEMS = C_STEPS * L

_mesh = plsc.VectorSubcoreMesh(core_axis_name="c", subcore_axis_name="s")


_GDN = lax.GatherDimensionNumbers(
    offset_dims=(), collapsed_slice_dims=(0,), start_index_map=(0,))


def _perm(x, idx):
    return lax.gather(x, idx[:, None], _GDN, slice_sizes=(1,),
                      mode=lax.GatherScatterMode.PROMISE_IN_BOUNDS)


def _lane_sum(v):
    lane = lax.iota(jnp.int32, L)
    for k in (8, 4, 2, 1):
        v = v + _perm(v, jnp.bitwise_xor(lane, k))
    return v


def _softmax_scatter(raw_ref, tab_ref, n_vregs, n_valid):
    lane = lax.iota(jnp.int32, L)
    es = []
    esum = jnp.zeros((L,), jnp.float32)
    for i in range(n_vregs):
        v = raw_ref[pl.ds(i * L, L)]
        valid = lane + (i * L) < n_valid
        e = jnp.where(valid, jnp.exp(v), jnp.zeros((L,), jnp.float32))
        es.append((e, valid))
        esum = esum + e
    r = 1.0 / _lane_sum(esum)
    for i, (e, valid) in enumerate(es):
        plsc.store_scatter(tab_ref, [lane + (i * L + 1)], e * r, mask=valid)


@functools.partial(
    pl.kernel,
    out_type=(
        jax.ShapeDtypeStruct((N_FLAT,), jnp.float32),
        jax.ShapeDtypeStruct((L_FLAT,), jnp.float32),
    ),
    mesh=_mesh,
    scratch_types=[
        pltpu.VMEM((64,), jnp.float32),
        pltpu.VMEM((16,), jnp.float32),
        pltpu.VMEM((64,), jnp.float32),
        pltpu.VMEM((16,), jnp.float32),
        pltpu.VMEM((N_PER_W,), jnp.int32),
        pltpu.VMEM((N_PER_W,), jnp.float32),
        pltpu.VMEM((L_PER_W,), jnp.int32),
        pltpu.VMEM((L_PER_W,), jnp.float32),
        pltpu.SemaphoreType.DMA,
        (pltpu.SemaphoreType.DMA,) * N_CHUNKS,
        pltpu.SemaphoreType.DMA,
        pltpu.SemaphoreType.DMA,
    ],
    compiler_params=pltpu.CompilerParams(needs_layout_passes=False),
)
def _gather_probs(gn_hbm, lk_hbm, ln_hbm, ll_hbm, out_n_hbm, out_l_hbm,
                  nraw, lraw, ntab, ltab, nidx, nout, lidx, lout,
                  sem_t, sem_i, sem_li, sem_o):
    wid = lax.axis_index("s") * NC + lax.axis_index("c")
    nbase = wid * N_PER_W
    lbase = wid * L_PER_W

    ln_c = pltpu.async_copy(ln_hbm, nraw.at[pl.ds(0, 49)], sem_t)
    ll_c = pltpu.async_copy(ll_hbm, lraw.at[pl.ds(0, 10)], sem_t)
    ni_c = [
        pltpu.async_copy(
            gn_hbm.at[pl.ds(nbase + c * C_ELEMS, C_ELEMS)],
            nidx.at[pl.ds(c * C_ELEMS, C_ELEMS)], sem_i[c])
        for c in range(N_CHUNKS)
    ]
    li_c = pltpu.async_copy(lk_hbm.at[pl.ds(lbase, L_PER_W)], lidx, sem_li)

    ln_c.wait()
    ll_c.wait()
    _softmax_scatter(nraw, ntab, 4, 49)
    _softmax_scatter(lraw, ltab, 1, 10)

    out_copies = []
    for c in range(N_CHUNKS):
        ni_c[c].wait()
        for i in range(c * C_STEPS, (c + 1) * C_STEPS):
            idx = nidx[pl.ds(i * L, L)]
            nout[pl.ds(i * L, L)] = plsc.load_gather(ntab, [idx])
        out_copies.append(pltpu.async_copy(
            nout.at[pl.ds(c * C_ELEMS, C_ELEMS)],
            out_n_hbm.at[pl.ds(nbase + c * C_ELEMS, C_ELEMS)], sem_o))

    li_c.wait()
    for i in range(L_STEPS):
        idx = lidx[pl.ds(i * L, L)]
        lout[pl.ds(i * L, L)] = plsc.load_gather(ltab, [idx])
    out_copies.append(
        pltpu.async_copy(lout, out_l_hbm.at[pl.ds(lbase, L_PER_W)], sem_o))

    for c in out_copies:
        c.wait()


def kernel(good_normal, lucky, log_normal_probs, log_lucky_probs):
    out_n, out_l = _gather_probs(
        good_normal.reshape(-1), lucky.reshape(-1),
        log_normal_probs, log_lucky_probs)
    return out_n.reshape(B, 5), out_l.reshape(B, 1)
